# Initial kernel scaffold; baseline (speedup 1.0000x reference)
import jax
import jax.numpy as jnp
from jax.experimental import pallas as pl


def kernel(pose):
    raise NotImplementedError("write your pallas kernel here")


# speedup vs baseline: 2.0931x; 2.0931x over previous
import math

import jax
import jax.numpy as jnp
from jax.experimental import pallas as pl
from jax.experimental.pallas import tpu as pltpu

_NBINS = 16
_B = 32
_T = 4096
_N = _T - 2
_WIN_MEAN = 16
_WIN_MAX = 8
_RAD2DEG = 180.0 / math.pi


def _shl(x, k):
    if k == 0:
        return x
    pad = jnp.zeros(x.shape[:-1] + (k,), x.dtype)
    return jnp.concatenate([x[..., k:], pad], axis=-1)


def _edge_shl(x):
    return jnp.concatenate([x[:, 1:], x[:, -1:]], axis=1)


def _rhythm_kernel(x_ref, peak_ref, env_ref, acc_ref):
    j = pl.program_id(0)

    @pl.when(j == 0)
    def _():
        acc_ref[...] = jnp.zeros_like(acc_ref)

    px = x_ref[0, 0]
    py = x_ref[0, 1]
    mx = _edge_shl(px) - px
    my = _edge_shl(py) - py
    mag = jnp.sqrt(mx * mx + my * my)
    phase = jnp.arctan2(my, mx)
    deg = (phase * _RAD2DEG) % 180.0
    bins = jnp.floor(deg).astype(jnp.int32) % _NBINS
    for b in range(_NBINS):
        acc_ref[b] = acc_ref[b] + jnp.where(bins == b, mag, 0.0)

    @pl.when(j == pl.num_programs(0) - 1)
    def _():
        rhy = jnp.zeros((_B, _T), jnp.float32)
        for b in range(_NBINS):
            d = acc_ref[b]
            sf = _edge_shl(d) - d
            rhy = rhy + jnp.maximum(sf, 0.0)
        env = rhy / jnp.max(rhy, axis=1, keepdims=True)
        gm = jnp.sum(env, axis=1, keepdims=True) / float(_N)

        t_idx = jax.lax.broadcasted_iota(jnp.int32, (_B, _T), 1)
        ssum = env
        for k in range(1, _WIN_MEAN):
            ssum = ssum + _shl(env, k)
        lm = jnp.where(t_idx <= _N - _WIN_MEAN, ssum / float(_WIN_MEAN), 0.0)

        m = jnp.maximum(env, _shl(env, 1))
        m = jnp.maximum(m, _shl(m, 2))
        m = jnp.maximum(m, _shl(m, 4))
        lx = jnp.where(t_idx <= _N - _WIN_MAX, m, 0.0)

        peak = ((lx - lm > 0.1 * gm) & (lx == env)).astype(jnp.int32)
        dmask = (_shl(peak, 1) - peak) != 0
        dmask = dmask & (t_idx < _N - 1)
        peak_ref[...] = peak * dmask.astype(jnp.int32)
        env_ref[...] = env


@jax.jit
def _run(gxy):
    peak, env = pl.pallas_call(
        _rhythm_kernel,
        grid=(17,),
        in_specs=[pl.BlockSpec((1, 2, _B, _T), lambda j: (j, 0, 0, 0))],
        out_specs=[
            pl.BlockSpec((_B, _T), lambda j: (0, 0)),
            pl.BlockSpec((_B, _T), lambda j: (0, 0)),
        ],
        out_shape=[
            jax.ShapeDtypeStruct((_B, _T), jnp.int32),
            jax.ShapeDtypeStruct((_B, _T), jnp.float32),
        ],
        scratch_shapes=[pltpu.VMEM((_NBINS, _B, _T), jnp.float32)],
    )(gxy)
    return peak[:, :_N], env[:, :_N, None]


def kernel(pose):
    gxy = jnp.transpose(pose[:, :, :, :2], (2, 3, 0, 1))
    return _run(gxy)
